# baseline (speedup 1.0000x reference)
import functools

import jax
import jax.numpy as jnp
from jax import lax
from jax.experimental import pallas as pl
from jax.experimental.pallas import tpu as pltpu
from jax.experimental.pallas import tpu_sc as plsc

_B = 16384
_D = 16
_NC = 2
_NS = 16
_NW = _NC * _NS
_BPW = _B // _NW
_CHUNK = 128
_NCHUNK = _BPW // _CHUNK

_BLK = 1024


def _gather_body(user_h, pos_h, neg_h, mfu_t, mfi_t, mlu_t, mli_t,
                 o_mfu, o_mfp, o_mfn, o_mlu, o_mlp, o_mln,
                 uidx, pidx, nidx,
                 r_mfu, r_mfp, r_mfn, r_mlu, r_mlp, r_mln, sem):
    wid = lax.axis_index("s") * _NC + lax.axis_index("c")
    base = wid * _BPW
    pltpu.sync_copy(user_h.at[pl.ds(base, _BPW)], uidx)
    pltpu.sync_copy(pos_h.at[pl.ds(base, _BPW)], pidx)
    pltpu.sync_copy(neg_h.at[pl.ds(base, _BPW)], nidx)
    copies = []
    for j in range(_NCHUNK):
        s = pl.ds(j * _CHUNK, _CHUNK)
        for tab, idx, dst in ((mfu_t, uidx, r_mfu), (mlu_t, uidx, r_mlu),
                              (mfi_t, pidx, r_mfp), (mli_t, pidx, r_mlp),
                              (mfi_t, nidx, r_mfn), (mli_t, nidx, r_mln)):
            copies.append(pltpu.async_copy(tab.at[idx.at[s]], dst.at[s], sem))
    for c in copies:
        c.wait()
    for src, dst in ((r_mfu, o_mfu), (r_mfp, o_mfp), (r_mfn, o_mfn),
                     (r_mlu, o_mlu), (r_mlp, o_mlp), (r_mln, o_mln)):
        pltpu.sync_copy(src, dst.at[pl.ds(base, _BPW)])


@jax.jit
def _gather6(user, pos, neg, mfu_t, mfi_t, mlu_t, mli_t):
    mesh = plsc.VectorSubcoreMesh(core_axis_name="c", subcore_axis_name="s")
    out = jax.ShapeDtypeStruct((_B, _D), jnp.float32)
    f = pl.kernel(
        _gather_body,
        out_type=[out] * 6,
        mesh=mesh,
        compiler_params=pltpu.CompilerParams(use_tc_tiling_on_sc=False),
        scratch_types=(
            [pltpu.VMEM((_BPW,), jnp.int32)] * 3
            + [pltpu.VMEM((_BPW, _D), jnp.float32)] * 6
            + [pltpu.SemaphoreType.DMA]
        ),
    )
    return f(user, pos, neg, mfu_t, mfi_t, mlu_t, mli_t)


def _tower_body(mfu, mfp, mfn, mlu, mlpos, mlneg,
                w1u, w1i, b1, w2, b2, w3, b3, w4, b4, wdm, wdl, bd, out):
    f32 = jnp.float32
    xu = jnp.dot(mlu[...], w1u[...], preferred_element_type=f32)
    hp = jnp.maximum(xu + jnp.dot(mlpos[...], w1i[...],
                                  preferred_element_type=f32) + b1[...], 0.0)
    hn = jnp.maximum(xu + jnp.dot(mlneg[...], w1i[...],
                                  preferred_element_type=f32) + b1[...], 0.0)
    for w, b in ((w2, b2), (w3, b3), (w4, b4)):
        hp = jnp.maximum(jnp.dot(hp, w[...], preferred_element_type=f32) + b[...], 0.0)
        hn = jnp.maximum(jnp.dot(hn, w[...], preferred_element_type=f32) + b[...], 0.0)
    mfp_v = jax.nn.sigmoid(mfu[...] * mfp[...])
    mfn_v = jax.nn.sigmoid(mfu[...] * mfn[...])
    sp = (jnp.dot(mfp_v, wdm[...], preferred_element_type=f32)
          + jnp.dot(hp, wdl[...], preferred_element_type=f32) + bd[0, 0])
    sn = (jnp.dot(mfn_v, wdm[...], preferred_element_type=f32)
          + jnp.dot(hn, wdl[...], preferred_element_type=f32) + bd[0, 0])
    out[...] = jnp.concatenate([sp, sn], axis=1)


def _tower(mfu, mfp, mfn, mlu, mlpos, mlneg,
           w1u, w1i, b1, w2, b2, w3, b3, w4, b4, wdm, wdl, bd,
           interpret=False):
    bspec = pl.BlockSpec((_BLK, _D), lambda i: (i, 0))

    def _full(a):
        return pl.BlockSpec(a.shape, lambda i: (0,) * a.ndim)

    weights = (w1u, w1i, b1, w2, b2, w3, b3, w4, b4, wdm, wdl, bd)
    return pl.pallas_call(
        _tower_body,
        grid=(_B // _BLK,),
        in_specs=[bspec] * 6 + [_full(w) for w in weights],
        out_specs=pl.BlockSpec((_BLK, 2), lambda i: (i, 0)),
        out_shape=jax.ShapeDtypeStruct((_B, 2), jnp.float32),
        interpret=interpret,
    )(mfu, mfp, mfn, mlu, mlpos, mlneg, *weights)


def kernel(user, pos_item, neg_item,
           mf_user_table, mf_item_table, mlp_user_table, mlp_item_table,
           W1, b1, W2, b2, W3, b3, W4, b4, Wd, bd):
    user = user.astype(jnp.int32)
    pos = pos_item.astype(jnp.int32)
    neg = neg_item.reshape(-1).astype(jnp.int32)
    mfu, mfp, mfn, mlu, mlpos, mlneg = _gather6(
        user, pos, neg,
        mf_user_table, mf_item_table, mlp_user_table, mlp_item_table)
    logits = _tower(
        mfu, mfp, mfn, mlu, mlpos, mlneg,
        W1[:_D], W1[_D:], b1.reshape(1, -1),
        W2, b2.reshape(1, -1), W3, b3.reshape(1, -1), W4, b4.reshape(1, -1),
        Wd[:_D], Wd[_D:], bd.reshape(1, 1))
    return logits
